# baseline (speedup 1.0000x reference)
import functools
import math

import jax
import jax.numpy as jnp
from jax import lax
from jax.experimental import pallas as pl
from jax.experimental.pallas import tpu as pltpu
from jax.experimental.pallas import tpu_sc as plsc

N_NODE = 10000
IN_DIM = 128
OUT_DIM = 128
H = 8
DK = OUT_DIM // H
E = 160000
_ROWS = 1000
_C = 64
_TILES = 16
_NPAD = 10240
_EPAD = 163840
_EPT = _EPAD // _TILES
_NCHUNK = _EPT // _C
_RPT = _NPAD // _TILES

_IDX_B = _C * 4
_ROW_B = _C * OUT_DIM * 4
_EXD_B = _C * H * 4


def _block_diag(a):
    eye = jnp.eye(H, dtype=a.dtype)
    return jnp.einsum('hij,hg->higj', a, eye).reshape(H * DK, H * DK)


def _qkv_body(x_ref, w_ref, b_ref, o_ref):
    o_ref[0] = (jnp.dot(x_ref[0], w_ref[0], preferred_element_type=jnp.float32)
                + b_ref[0])


def _qkv_project(x, w, b):
    n = x.shape[1]
    return pl.pallas_call(
        _qkv_body,
        grid=(2, n // _ROWS),
        in_specs=[
            pl.BlockSpec((1, _ROWS, IN_DIM), lambda i, j: (i, j, 0)),
            pl.BlockSpec((1, IN_DIM, 3 * OUT_DIM), lambda i, j: (i, 0, 0)),
            pl.BlockSpec((1, 1, 3 * OUT_DIM), lambda i, j: (i, 0, 0)),
        ],
        out_specs=pl.BlockSpec((1, _ROWS, 3 * OUT_DIM), lambda i, j: (i, j, 0)),
        out_shape=jax.ShapeDtypeStruct((2, n, 3 * OUT_DIM), jnp.float32),
    )(x, w, b)


def _epi_body(tp_ref, den_ref, h_ref, w_ref, b_ref, hc_ref, g_ref, bb_ref,
              ex_ref, o_ref):
    recip = 1.0 / (den_ref[0] + 1e-9)
    recip128 = jnp.dot(recip, ex_ref[...], preferred_element_type=jnp.float32)
    t = tp_ref[0] * recip128
    out = (jnp.dot(t, w_ref[0], preferred_element_type=jnp.float32)
           + b_ref[0] + h_ref[0] * hc_ref[0])
    mu = jnp.mean(out, axis=-1, keepdims=True)
    var = jnp.mean((out - mu) ** 2, axis=-1, keepdims=True)
    o_ref[0] = (out - mu) / jnp.sqrt(var + 1e-5) * g_ref[0] + bb_ref[0]


def _epilogue(tp, den, h, w, b, hc, g, bb, expand):
    n = h.shape[1]
    vec = lambda: pl.BlockSpec((1, 1, OUT_DIM), lambda i, j: (i, 0, 0))
    return pl.pallas_call(
        _epi_body,
        grid=(2, n // _ROWS),
        in_specs=[
            pl.BlockSpec((1, _ROWS, OUT_DIM), lambda i, j: (i, j, 0)),
            pl.BlockSpec((1, _ROWS, H), lambda i, j: (i, j, 0)),
            pl.BlockSpec((1, _ROWS, OUT_DIM), lambda i, j: (i, j, 0)),
            pl.BlockSpec((1, OUT_DIM, OUT_DIM), lambda i, j: (i, 0, 0)),
            vec(), vec(), vec(), vec(),
            pl.BlockSpec((H, OUT_DIM), lambda i, j: (0, 0)),
        ],
        out_specs=pl.BlockSpec((1, _ROWS, OUT_DIM), lambda i, j: (i, j, 0)),
        out_shape=jax.ShapeDtypeStruct((2, n, OUT_DIM), jnp.float32),
    )(tp, den, h, w, b, hc, g, bb, expand)


def _sc_body(q2, k2, v2, src2, dst2, tp_out, den_out,
             krows0, krows1, qm0, qm1, vrows, exd0, exd1,
             srcl0, srcl1, dstl0, dstl1, srcg0, srcg1,
             dstg0, dstg1, dsts0, dsts1, rottab_ref,
             t_acc, den_acc, gsem, vsem, isem, ssem):
    c = lax.axis_index("c")
    s = lax.axis_index("s")
    zeros16 = jnp.zeros((16,), jnp.float32)
    iota16 = lax.iota(jnp.int32, 16)
    krows = (krows0, krows1)
    qm = (qm0, qm1)
    exd = (exd0, exd1)
    srcl = (srcl0, srcl1)
    dstl = (dstl0, dstl1)
    srcg = (srcg0, srcg1)
    dstg = (dstg0, dstg1)
    dsts = (dsts0, dsts1)
    rottab = rottab_ref

    koff = c * _NPAD
    qoff = (1 - c) * _NPAD
    ebase = c * _EPAD + s * _EPT
    r0 = s * _RPT

    def zbuf_body(r, _):
        for j in range(OUT_DIM // 16):
            qm1[r, pl.ds(j * 16, 16)] = zeros16
        return 0
    lax.fori_loop(0, _C, zbuf_body, 0)

    def zdsts_body(r, _):
        dsts1[pl.ds(r * 16, 16)] = jnp.zeros((16,), jnp.int32)
        return 0
    lax.fori_loop(0, _C // 16, zdsts_body, 0)

    for d in range(DK):
        rottab_ref[d] = (iota16 + d) & 15

    def zexd2_body(g, _):
        cols = jnp.zeros((16,), jnp.float32)
        rows = iota16 + g * 16
        for h in range(H):
            plsc.store_scatter(exd1, [rows, jnp.full((16,), h, jnp.int32)],
                               cols)
        return 0
    lax.fori_loop(0, _C // 16, zexd2_body, 0)

    for kblk in range(_RPT // _C):
        pltpu.sync_copy(qm1, t_acc.at[pl.ds(r0 + kblk * _C, _C)])
        pltpu.sync_copy(exd1, den_acc.at[pl.ds(r0 + kblk * _C, _C)])
    plsc.subcore_barrier()

    def adjust(dst_sg, dst_dg, src_sl, src_dl):
        for j in range(_C // 16):
            sl = pl.ds(j * 16, 16)
            dst_sg[sl] = src_sl[sl] + koff
            dst_dg[sl] = src_dl[sl] + qoff

    pltpu.sync_copy(src2.at[pl.ds(ebase, _C)], srcl0)
    pltpu.sync_copy(dst2.at[pl.ds(ebase, _C)], dstl0)
    adjust(srcg0, dstg0, srcl0, dstl0)
    pltpu.async_copy(k2.at[srcg0], krows0, gsem)
    pltpu.async_copy(q2.at[dstg0], qm0, gsem)
    pltpu.async_copy(src2.at[pl.ds(ebase + _C, _C)], srcl1, isem)
    pltpu.async_copy(dst2.at[pl.ds(ebase + _C, _C)], dstl1, isem)
    pltpu.async_copy(qm1, t_acc.at[dsts1], ssem, add=True)
    pltpu.async_copy(exd1, den_acc.at[dsts1], ssem, add=True)

    def score_phase(kr, qr, xd):
        @plsc.parallel_loop(0, _C // 16, unroll=2)
        def _score_groups(g):
            rows = iota16 + g * 16
            for h in range(H):
                acc0 = zeros16
                acc1 = zeros16
                for d in range(DK):
                    col = rottab[d] + h * DK
                    qv = plsc.load_gather(qr, [rows, col])
                    kv = plsc.load_gather(kr, [rows, col])
                    if d % 2 == 0:
                        acc0 = acc0 + qv * kv
                    else:
                        acc1 = acc1 + qv * kv
                ex = jnp.exp(jnp.minimum(acc0 + acc1, 60.0))
                plsc.store_scatter(xd, [rows, jnp.full((16,), h, jnp.int32)],
                                   ex)

    def m_phase(qr, xd):
        @plsc.parallel_loop(0, _C // 16, unroll=2)
        def _m_groups(g):
            rows = iota16 + g * 16
            for h in range(H):
                ex = plsc.load_gather(xd, [rows, jnp.full((16,), h, jnp.int32)])
                for d in range(DK):
                    col = rottab[d] + h * DK
                    vv = plsc.load_gather(vrows, [rows, col])
                    plsc.store_scatter(qr, [rows, col], vv * ex)

    def drain(sem, src, dst):
        pltpu.make_async_copy(src, dst, sem).wait()

    def pair_body(i, _):
        for b in range(2):
            ci = i * 2 + b
            o = 1 - b
            drain(isem, src2.at[pl.ds(0, _C)], srcl[o])
            drain(isem, dst2.at[pl.ds(0, _C)], dstl[o])
            adjust(srcg[o], dstg[o], srcl[o], dstl[o])
            drain(ssem, k2.at[pl.ds(0, _C)], qm[o])
            drain(ssem, den_out.at[pl.ds(0, _C)], exd[o])
            drain(gsem, k2.at[pl.ds(0, _C)], krows[b])
            drain(gsem, q2.at[pl.ds(0, _C)], qm[b])
            pltpu.async_copy(k2.at[srcg[o]], krows[o], gsem)
            pltpu.async_copy(q2.at[dstg[o]], qm[o], gsem)
            for j in range(_C // 16):
                sl = pl.ds(j * 16, 16)
                dsts[b][sl] = dstl[b][sl]
            base2 = ebase + (ci + 2) * _C
            pltpu.async_copy(src2.at[pl.ds(base2, _C)], srcl[b], isem)
            pltpu.async_copy(dst2.at[pl.ds(base2, _C)], dstl[b], isem)
            pltpu.async_copy(v2.at[srcg[b]], vrows, vsem)
            score_phase(krows[b], qm[b], exd[b])
            drain(vsem, v2.at[pl.ds(0, _C)], vrows)
            m_phase(qm[b], exd[b])
            pltpu.async_copy(qm[b], t_acc.at[dsts[b]], ssem, add=True)
            pltpu.async_copy(exd[b], den_acc.at[dsts[b]], ssem, add=True)
        return 0

    lax.fori_loop(0, _NCHUNK // 2, pair_body, 0)

    drain(ssem, k2.at[pl.ds(0, _C)], qm1)
    drain(ssem, den_out.at[pl.ds(0, _C)], exd1)
    drain(gsem, k2.at[pl.ds(0, _C)], krows0)
    drain(gsem, q2.at[pl.ds(0, _C)], qm0)
    drain(isem, src2.at[pl.ds(0, _C)], srcl0)
    drain(isem, dst2.at[pl.ds(0, _C)], dstl0)
    plsc.subcore_barrier()

    out_base = c * _NPAD + r0
    pltpu.sync_copy(t_acc.at[pl.ds(r0, _RPT)],
                    tp_out.at[pl.ds(out_base, _RPT)])
    pltpu.sync_copy(den_acc.at[pl.ds(r0, _RPT)],
                    den_out.at[pl.ds(out_base, _RPT)])


def _sc_middle(q2, k2, v2, src2, dst2):
    mesh = plsc.VectorSubcoreMesh(core_axis_name="c", subcore_axis_name="s")
    f = pl.kernel(
        _sc_body,
        mesh=mesh,
        compiler_params=pltpu.CompilerParams(needs_layout_passes=False,
                                             use_tc_tiling_on_sc=False),
        out_type=[
            jax.ShapeDtypeStruct((2 * _NPAD, OUT_DIM), jnp.float32),
            jax.ShapeDtypeStruct((2 * _NPAD, H), jnp.float32),
        ],
        scratch_types=[
            pltpu.VMEM((_C, OUT_DIM), jnp.float32),
            pltpu.VMEM((_C, OUT_DIM), jnp.float32),
            pltpu.VMEM((_C, OUT_DIM), jnp.float32),
            pltpu.VMEM((_C, OUT_DIM), jnp.float32),
            pltpu.VMEM((_C, OUT_DIM), jnp.float32),
            pltpu.VMEM((_C, H), jnp.float32),
            pltpu.VMEM((_C, H), jnp.float32),
            pltpu.VMEM((_C,), jnp.int32),
            pltpu.VMEM((_C,), jnp.int32),
            pltpu.VMEM((_C,), jnp.int32),
            pltpu.VMEM((_C,), jnp.int32),
            pltpu.VMEM((_C,), jnp.int32),
            pltpu.VMEM((_C,), jnp.int32),
            pltpu.VMEM((_C,), jnp.int32),
            pltpu.VMEM((_C,), jnp.int32),
            pltpu.VMEM((_C,), jnp.int32),
            pltpu.VMEM((_C,), jnp.int32),
            pltpu.VMEM((DK, 16), jnp.int32),
            pltpu.VMEM_SHARED((_NPAD, OUT_DIM), jnp.float32),
            pltpu.VMEM_SHARED((_NPAD, H), jnp.float32),
            pltpu.SemaphoreType.DMA,
            pltpu.SemaphoreType.DMA,
            pltpu.SemaphoreType.DMA,
            pltpu.SemaphoreType.DMA,
        ],
    )
    return f(q2, k2, v2, src2, dst2)


def kernel(h_author, h_paper, edge_index_writes, edge_index_written_by,
           Wk_author, bk_author, Wq_author, bq_author, Wv_author, bv_author,
           Wa_author, ba_author, ln_g_author, ln_b_author, skip_author,
           Wk_paper, bk_paper, Wq_paper, bq_paper, Wv_paper, bv_paper,
           Wa_paper, ba_paper, ln_g_paper, ln_b_paper, skip_paper,
           pri_writes, att_writes, msg_writes,
           pri_written_by, att_written_by, msg_written_by):
    scale_w = pri_writes / math.sqrt(DK)
    scale_wb = pri_written_by / math.sqrt(DK)
    bd_att_w = _block_diag(att_writes * scale_w[:, None, None])
    bd_msg_w = _block_diag(msg_writes)
    bd_att_wb = _block_diag(att_written_by * scale_wb[:, None, None])
    bd_msg_wb = _block_diag(msg_written_by)
    w_author = jnp.concatenate(
        [Wq_author, Wk_author @ bd_att_w, Wv_author @ bd_msg_w], axis=1)
    b_author = jnp.concatenate(
        [bq_author, bk_author @ bd_att_w, bv_author @ bd_msg_w])[None, :]
    w_paper = jnp.concatenate(
        [Wq_paper, Wk_paper @ bd_att_wb, Wv_paper @ bd_msg_wb], axis=1)
    b_paper = jnp.concatenate(
        [bq_paper, bk_paper @ bd_att_wb, bv_paper @ bd_msg_wb])[None, :]

    x = jnp.stack([h_author, h_paper])
    w = jnp.stack([w_author, w_paper])
    b = jnp.stack([b_author, b_paper])

    qkv = _qkv_project(x, w, b)
    qkv_p = jnp.pad(qkv, ((0, 0), (0, _NPAD - N_NODE), (0, 0)))
    q2 = qkv_p[:, :, :OUT_DIM].reshape(2 * _NPAD, OUT_DIM)
    k2 = qkv_p[:, :, OUT_DIM:2 * OUT_DIM].reshape(2 * _NPAD, OUT_DIM)
    v2 = qkv_p[:, :, 2 * OUT_DIM:].reshape(2 * _NPAD, OUT_DIM)

    npad_e = _EPAD - E
    pad_src = jnp.zeros((npad_e,), jnp.int32)
    pad_dst = jnp.full((npad_e,), N_NODE, jnp.int32)
    tail = jnp.zeros((4 * _C,), jnp.int32)
    src2 = jnp.concatenate([edge_index_writes[0], pad_src,
                            edge_index_written_by[0], pad_src, tail])
    dst2 = jnp.concatenate([edge_index_writes[1], pad_dst,
                            edge_index_written_by[1], pad_dst, tail])

    tp, den = _sc_middle(q2, k2, v2, src2, dst2)
    tp_s = jnp.stack([tp[_NPAD:_NPAD + N_NODE], tp[:N_NODE]])
    den_s = jnp.stack([den[_NPAD:_NPAD + N_NODE], den[:N_NODE]])

    alpha_a = jax.nn.sigmoid(skip_author)[0]
    alpha_p = jax.nn.sigmoid(skip_paper)[0]
    w2 = jnp.stack([Wa_author * alpha_a, Wa_paper * alpha_p])
    b2 = jnp.stack([(ba_author * alpha_a)[None, :], (ba_paper * alpha_p)[None, :]])
    hc = jnp.stack([jnp.full((1, OUT_DIM), 1.0 - alpha_a, jnp.float32),
                    jnp.full((1, OUT_DIM), 1.0 - alpha_p, jnp.float32)])
    g2 = jnp.stack([ln_g_author[None, :], ln_g_paper[None, :]])
    bb2 = jnp.stack([ln_b_author[None, :], ln_b_paper[None, :]])
    expand = jnp.kron(jnp.eye(H, dtype=jnp.float32),
                      jnp.ones((1, DK), jnp.float32))
    hstack = jnp.stack([h_author, h_paper])
    out = _epilogue(tp_s, den_s, hstack, w2, b2, hc, g2, bb2, expand)
    return (out[0], out[1])


# speedup vs baseline: 2.5847x; 2.5847x over previous
import functools
import math

import jax
import jax.numpy as jnp
from jax import lax
from jax.experimental import pallas as pl
from jax.experimental.pallas import tpu as pltpu
from jax.experimental.pallas import tpu_sc as plsc

N_NODE = 10000
IN_DIM = 128
OUT_DIM = 128
H = 8
DK = OUT_DIM // H
E = 160000
_ROWS = 1000
_C = 64
_TILES = 16
_NPAD = 10240
_EPAD = 163840
_EPT = _EPAD // _TILES
_NCHUNK = _EPT // _C
_RPT = _NPAD // _TILES

_IDX_B = _C * 4
_ROW_B = _C * OUT_DIM * 4
_EXD_B = _C * H * 4


def _block_diag(a):
    eye = jnp.eye(H, dtype=a.dtype)
    return jnp.einsum('hij,hg->higj', a, eye).reshape(H * DK, H * DK)


def _qkv_body(x_ref, w_ref, b_ref, o_ref):
    o_ref[0] = (jnp.dot(x_ref[0], w_ref[0], preferred_element_type=jnp.float32)
                + b_ref[0])


def _qkv_project(x, w, b):
    n = x.shape[1]
    return pl.pallas_call(
        _qkv_body,
        grid=(2, n // _ROWS),
        in_specs=[
            pl.BlockSpec((1, _ROWS, IN_DIM), lambda i, j: (i, j, 0)),
            pl.BlockSpec((1, IN_DIM, 3 * OUT_DIM), lambda i, j: (i, 0, 0)),
            pl.BlockSpec((1, 1, 3 * OUT_DIM), lambda i, j: (i, 0, 0)),
        ],
        out_specs=pl.BlockSpec((1, _ROWS, 3 * OUT_DIM), lambda i, j: (i, j, 0)),
        out_shape=jax.ShapeDtypeStruct((2, n, 3 * OUT_DIM), jnp.float32),
    )(x, w, b)


def _epi_body(tp_ref, den_ref, h_ref, w_ref, b_ref, hc_ref, g_ref, bb_ref,
              ex_ref, o_ref):
    recip = 1.0 / (den_ref[0] + 1e-9)
    recip128 = jnp.dot(recip, ex_ref[...], preferred_element_type=jnp.float32)
    t = tp_ref[0] * recip128
    out = (jnp.dot(t, w_ref[0], preferred_element_type=jnp.float32)
           + b_ref[0] + h_ref[0] * hc_ref[0])
    mu = jnp.mean(out, axis=-1, keepdims=True)
    var = jnp.mean((out - mu) ** 2, axis=-1, keepdims=True)
    o_ref[0] = (out - mu) / jnp.sqrt(var + 1e-5) * g_ref[0] + bb_ref[0]


def _epilogue(tp, den, h, w, b, hc, g, bb, expand):
    n = h.shape[1]
    vec = lambda: pl.BlockSpec((1, 1, OUT_DIM), lambda i, j: (i, 0, 0))
    return pl.pallas_call(
        _epi_body,
        grid=(2, n // _ROWS),
        in_specs=[
            pl.BlockSpec((1, _ROWS, OUT_DIM), lambda i, j: (i, j, 0)),
            pl.BlockSpec((1, _ROWS, H), lambda i, j: (i, j, 0)),
            pl.BlockSpec((1, _ROWS, OUT_DIM), lambda i, j: (i, j, 0)),
            pl.BlockSpec((1, OUT_DIM, OUT_DIM), lambda i, j: (i, 0, 0)),
            vec(), vec(), vec(), vec(),
            pl.BlockSpec((H, OUT_DIM), lambda i, j: (0, 0)),
        ],
        out_specs=pl.BlockSpec((1, _ROWS, OUT_DIM), lambda i, j: (i, j, 0)),
        out_shape=jax.ShapeDtypeStruct((2, n, OUT_DIM), jnp.float32),
    )(tp, den, h, w, b, hc, g, bb, expand)


def _sc_body(q2, k2, v2, src2, dst2, tp_out, den_out,
             krows0, krows1, qm0, qm1, vrows, exd0, exd1,
             srcl0, srcl1, dstl0, dstl1, srcg0, srcg1,
             dstg0, dstg1, dsts0, dsts1, rottab_ref,
             t_acc, den_acc, gsem, vsem, isem, ssem):
    c = lax.axis_index("c")
    s = lax.axis_index("s")
    zeros16 = jnp.zeros((16,), jnp.float32)
    iota16 = lax.iota(jnp.int32, 16)
    krows = (krows0, krows1)
    qm = (qm0, qm1)
    exd = (exd0, exd1)
    srcl = (srcl0, srcl1)
    dstl = (dstl0, dstl1)
    srcg = (srcg0, srcg1)
    dstg = (dstg0, dstg1)
    dsts = (dsts0, dsts1)
    rottab = rottab_ref

    koff = c * _NPAD
    qoff = (1 - c) * _NPAD
    ebase = c * _EPAD + s * _EPT
    r0 = s * _RPT

    def zbuf_body(r, _):
        for j in range(OUT_DIM // 16):
            qm1[r, pl.ds(j * 16, 16)] = zeros16
        return 0
    lax.fori_loop(0, _C, zbuf_body, 0)

    def zdsts_body(r, _):
        dsts1[pl.ds(r * 16, 16)] = jnp.zeros((16,), jnp.int32)
        return 0
    lax.fori_loop(0, _C // 16, zdsts_body, 0)

    for d in range(DK):
        rottab_ref[d] = (iota16 + d) & 15

    def zexd2_body(g, _):
        cols = jnp.zeros((16,), jnp.float32)
        rows = iota16 + g * 16
        for h in range(H):
            plsc.store_scatter(exd1, [rows, jnp.full((16,), h, jnp.int32)],
                               cols)
        return 0
    lax.fori_loop(0, _C // 16, zexd2_body, 0)

    for kblk in range(_RPT // _C):
        pltpu.sync_copy(qm1, t_acc.at[pl.ds(r0 + kblk * _C, _C)])
        pltpu.sync_copy(exd1, den_acc.at[pl.ds(r0 + kblk * _C, _C)])
    plsc.subcore_barrier()

    def adjust(dst_sg, dst_dg, src_sl, src_dl):
        for j in range(_C // 16):
            sl = pl.ds(j * 16, 16)
            dst_sg[sl] = src_sl[sl] + koff
            dst_dg[sl] = src_dl[sl] + qoff

    pltpu.sync_copy(src2.at[pl.ds(ebase, _C)], srcl0)
    pltpu.sync_copy(dst2.at[pl.ds(ebase, _C)], dstl0)
    adjust(srcg0, dstg0, srcl0, dstl0)
    pltpu.async_copy(k2.at[srcg0], krows0, gsem)
    pltpu.async_copy(q2.at[dstg0], qm0, gsem)
    pltpu.async_copy(src2.at[pl.ds(ebase + _C, _C)], srcl1, isem)
    pltpu.async_copy(dst2.at[pl.ds(ebase + _C, _C)], dstl1, isem)
    pltpu.async_copy(qm1, t_acc.at[dsts1], ssem, add=True)
    pltpu.async_copy(exd1, den_acc.at[dsts1], ssem, add=True)

    def score_phase(kr, qr, xd):
        @plsc.parallel_loop(0, _C // 16, unroll=1)
        def _score_groups(g):
            rows = iota16 + g * 16
            for h in range(H):
                acc0 = zeros16
                acc1 = zeros16
                for half in range(2):
                    qvs = []
                    kvs = []
                    for dd in range(DK // 2):
                        d = half * (DK // 2) + dd
                        col = rottab[d] + h * DK
                        qvs.append(plsc.load_gather(qr, [rows, col]))
                        kvs.append(plsc.load_gather(kr, [rows, col]))
                    for dd in range(DK // 2):
                        if dd % 2 == 0:
                            acc0 = acc0 + qvs[dd] * kvs[dd]
                        else:
                            acc1 = acc1 + qvs[dd] * kvs[dd]
                ex = jnp.exp(jnp.minimum(acc0 + acc1, 60.0))
                plsc.store_scatter(xd, [rows, jnp.full((16,), h, jnp.int32)],
                                   ex)

    def m_phase(qr, xd):
        @plsc.parallel_loop(0, _C, unroll=2)
        def _m_edges(e):
            erow = jnp.full((16,), e, jnp.int32)
            for h in range(H):
                exb = plsc.load_gather(xd, [erow,
                                            jnp.full((16,), h, jnp.int32)])
                vv = vrows[e, pl.ds(h * DK, DK)]
                qr[e, pl.ds(h * DK, DK)] = vv * exb

    def drain(sem, src, dst):
        pltpu.make_async_copy(src, dst, sem).wait()

    def pair_body(i, _):
        for b in range(2):
            ci = i * 2 + b
            o = 1 - b
            drain(isem, src2.at[pl.ds(0, _C)], srcl[o])
            drain(isem, dst2.at[pl.ds(0, _C)], dstl[o])
            adjust(srcg[o], dstg[o], srcl[o], dstl[o])
            drain(ssem, k2.at[pl.ds(0, _C)], qm[o])
            drain(ssem, den_out.at[pl.ds(0, _C)], exd[o])
            drain(gsem, k2.at[pl.ds(0, _C)], krows[b])
            drain(gsem, q2.at[pl.ds(0, _C)], qm[b])
            pltpu.async_copy(k2.at[srcg[o]], krows[o], gsem)
            pltpu.async_copy(q2.at[dstg[o]], qm[o], gsem)
            for j in range(_C // 16):
                sl = pl.ds(j * 16, 16)
                dsts[b][sl] = dstl[b][sl]
            base2 = ebase + (ci + 2) * _C
            pltpu.async_copy(src2.at[pl.ds(base2, _C)], srcl[b], isem)
            pltpu.async_copy(dst2.at[pl.ds(base2, _C)], dstl[b], isem)
            pltpu.async_copy(v2.at[srcg[b]], vrows, vsem)
            score_phase(krows[b], qm[b], exd[b])
            drain(vsem, v2.at[pl.ds(0, _C)], vrows)
            m_phase(qm[b], exd[b])
            pltpu.async_copy(qm[b], t_acc.at[dsts[b]], ssem, add=True)
            pltpu.async_copy(exd[b], den_acc.at[dsts[b]], ssem, add=True)
        return 0

    lax.fori_loop(0, _NCHUNK // 2, pair_body, 0)

    drain(ssem, k2.at[pl.ds(0, _C)], qm1)
    drain(ssem, den_out.at[pl.ds(0, _C)], exd1)
    drain(gsem, k2.at[pl.ds(0, _C)], krows0)
    drain(gsem, q2.at[pl.ds(0, _C)], qm0)
    drain(isem, src2.at[pl.ds(0, _C)], srcl0)
    drain(isem, dst2.at[pl.ds(0, _C)], dstl0)
    plsc.subcore_barrier()

    out_base = c * _NPAD + r0
    pltpu.sync_copy(t_acc.at[pl.ds(r0, _RPT)],
                    tp_out.at[pl.ds(out_base, _RPT)])
    pltpu.sync_copy(den_acc.at[pl.ds(r0, _RPT)],
                    den_out.at[pl.ds(out_base, _RPT)])


def _sc_middle(q2, k2, v2, src2, dst2):
    mesh = plsc.VectorSubcoreMesh(core_axis_name="c", subcore_axis_name="s")
    f = pl.kernel(
        _sc_body,
        mesh=mesh,
        compiler_params=pltpu.CompilerParams(needs_layout_passes=False,
                                             use_tc_tiling_on_sc=False),
        out_type=[
            jax.ShapeDtypeStruct((2 * _NPAD, OUT_DIM), jnp.float32),
            jax.ShapeDtypeStruct((2 * _NPAD, H), jnp.float32),
        ],
        scratch_types=[
            pltpu.VMEM((_C, OUT_DIM), jnp.float32),
            pltpu.VMEM((_C, OUT_DIM), jnp.float32),
            pltpu.VMEM((_C, OUT_DIM), jnp.float32),
            pltpu.VMEM((_C, OUT_DIM), jnp.float32),
            pltpu.VMEM((_C, OUT_DIM), jnp.float32),
            pltpu.VMEM((_C, H), jnp.float32),
            pltpu.VMEM((_C, H), jnp.float32),
            pltpu.VMEM((_C,), jnp.int32),
            pltpu.VMEM((_C,), jnp.int32),
            pltpu.VMEM((_C,), jnp.int32),
            pltpu.VMEM((_C,), jnp.int32),
            pltpu.VMEM((_C,), jnp.int32),
            pltpu.VMEM((_C,), jnp.int32),
            pltpu.VMEM((_C,), jnp.int32),
            pltpu.VMEM((_C,), jnp.int32),
            pltpu.VMEM((_C,), jnp.int32),
            pltpu.VMEM((_C,), jnp.int32),
            pltpu.VMEM((DK, 16), jnp.int32),
            pltpu.VMEM_SHARED((_NPAD, OUT_DIM), jnp.float32),
            pltpu.VMEM_SHARED((_NPAD, H), jnp.float32),
            pltpu.SemaphoreType.DMA,
            pltpu.SemaphoreType.DMA,
            pltpu.SemaphoreType.DMA,
            pltpu.SemaphoreType.DMA,
        ],
    )
    return f(q2, k2, v2, src2, dst2)


def kernel(h_author, h_paper, edge_index_writes, edge_index_written_by,
           Wk_author, bk_author, Wq_author, bq_author, Wv_author, bv_author,
           Wa_author, ba_author, ln_g_author, ln_b_author, skip_author,
           Wk_paper, bk_paper, Wq_paper, bq_paper, Wv_paper, bv_paper,
           Wa_paper, ba_paper, ln_g_paper, ln_b_paper, skip_paper,
           pri_writes, att_writes, msg_writes,
           pri_written_by, att_written_by, msg_written_by):
    scale_w = pri_writes / math.sqrt(DK)
    scale_wb = pri_written_by / math.sqrt(DK)
    bd_att_w = _block_diag(att_writes * scale_w[:, None, None])
    bd_msg_w = _block_diag(msg_writes)
    bd_att_wb = _block_diag(att_written_by * scale_wb[:, None, None])
    bd_msg_wb = _block_diag(msg_written_by)
    w_author = jnp.concatenate(
        [Wq_author, Wk_author @ bd_att_w, Wv_author @ bd_msg_w], axis=1)
    b_author = jnp.concatenate(
        [bq_author, bk_author @ bd_att_w, bv_author @ bd_msg_w])[None, :]
    w_paper = jnp.concatenate(
        [Wq_paper, Wk_paper @ bd_att_wb, Wv_paper @ bd_msg_wb], axis=1)
    b_paper = jnp.concatenate(
        [bq_paper, bk_paper @ bd_att_wb, bv_paper @ bd_msg_wb])[None, :]

    x = jnp.stack([h_author, h_paper])
    w = jnp.stack([w_author, w_paper])
    b = jnp.stack([b_author, b_paper])

    qkv = _qkv_project(x, w, b)
    qkv_p = jnp.pad(qkv, ((0, 0), (0, _NPAD - N_NODE), (0, 0)))
    q2 = qkv_p[:, :, :OUT_DIM].reshape(2 * _NPAD, OUT_DIM)
    k2 = qkv_p[:, :, OUT_DIM:2 * OUT_DIM].reshape(2 * _NPAD, OUT_DIM)
    v2 = qkv_p[:, :, 2 * OUT_DIM:].reshape(2 * _NPAD, OUT_DIM)

    npad_e = _EPAD - E
    pad_src = jnp.zeros((npad_e,), jnp.int32)
    pad_dst = jnp.full((npad_e,), N_NODE, jnp.int32)
    tail = jnp.zeros((4 * _C,), jnp.int32)
    src2 = jnp.concatenate([edge_index_writes[0], pad_src,
                            edge_index_written_by[0], pad_src, tail])
    dst2 = jnp.concatenate([edge_index_writes[1], pad_dst,
                            edge_index_written_by[1], pad_dst, tail])

    tp, den = _sc_middle(q2, k2, v2, src2, dst2)
    tp_s = jnp.stack([tp[_NPAD:_NPAD + N_NODE], tp[:N_NODE]])
    den_s = jnp.stack([den[_NPAD:_NPAD + N_NODE], den[:N_NODE]])

    alpha_a = jax.nn.sigmoid(skip_author)[0]
    alpha_p = jax.nn.sigmoid(skip_paper)[0]
    w2 = jnp.stack([Wa_author * alpha_a, Wa_paper * alpha_p])
    b2 = jnp.stack([(ba_author * alpha_a)[None, :], (ba_paper * alpha_p)[None, :]])
    hc = jnp.stack([jnp.full((1, OUT_DIM), 1.0 - alpha_a, jnp.float32),
                    jnp.full((1, OUT_DIM), 1.0 - alpha_p, jnp.float32)])
    g2 = jnp.stack([ln_g_author[None, :], ln_g_paper[None, :]])
    bb2 = jnp.stack([ln_b_author[None, :], ln_b_paper[None, :]])
    expand = jnp.kron(jnp.eye(H, dtype=jnp.float32),
                      jnp.ones((1, DK), jnp.float32))
    hstack = jnp.stack([h_author, h_paper])
    out = _epilogue(tp_s, den_s, hstack, w2, b2, hc, g2, bb2, expand)
    return (out[0], out[1])


# speedup vs baseline: 2.7124x; 1.0494x over previous
import functools
import math

import jax
import jax.numpy as jnp
from jax import lax
from jax.experimental import pallas as pl
from jax.experimental.pallas import tpu as pltpu
from jax.experimental.pallas import tpu_sc as plsc

N_NODE = 10000
IN_DIM = 128
OUT_DIM = 128
H = 8
DK = OUT_DIM // H
E = 160000
_ROWS = 1000
_C = 64
_TILES = 16
_NPAD = 10240
_EPAD = 163840
_EPT = _EPAD // _TILES
_NCHUNK = _EPT // _C
_RPT = _NPAD // _TILES

_IDX_B = _C * 4
_ROW_B = _C * OUT_DIM * 4
_EXD_B = _C * H * 4


def _block_diag(a):
    eye = jnp.eye(H, dtype=a.dtype)
    return jnp.einsum('hij,hg->higj', a, eye).reshape(H * DK, H * DK)


_QR = 1024


def _qkv_body(x_ref, w_ref, b_ref, oq_ref, ok_ref, ov_ref):
    res = (jnp.dot(x_ref[0], w_ref[0], preferred_element_type=jnp.float32)
           + b_ref[0])
    oq_ref[...] = res[:, :OUT_DIM]
    ok_ref[...] = res[:, OUT_DIM:2 * OUT_DIM]
    ov_ref[...] = res[:, 2 * OUT_DIM:]


def _qkv_project(x, w, b):
    out = lambda: pl.BlockSpec((_QR, OUT_DIM),
                               lambda i, j: (i * (_NPAD // _QR) + j, 0))
    shp = jax.ShapeDtypeStruct((2 * _NPAD, OUT_DIM), jnp.float32)
    return pl.pallas_call(
        _qkv_body,
        grid=(2, _NPAD // _QR),
        in_specs=[
            pl.BlockSpec((1, _QR, IN_DIM), lambda i, j: (i, j, 0)),
            pl.BlockSpec((1, IN_DIM, 3 * OUT_DIM), lambda i, j: (i, 0, 0)),
            pl.BlockSpec((1, 1, 3 * OUT_DIM), lambda i, j: (i, 0, 0)),
        ],
        out_specs=[out(), out(), out()],
        out_shape=[shp, shp, shp],
    )(x, w, b)


def _epi_body(tp_ref, den_ref, h_ref, w_ref, b_ref, hc_ref, g_ref, bb_ref,
              ex_ref, o_ref):
    recip = 1.0 / (den_ref[...] + 1e-9)
    recip128 = jnp.dot(recip, ex_ref[...], preferred_element_type=jnp.float32)
    t = tp_ref[...] * recip128
    out = (jnp.dot(t, w_ref[0], preferred_element_type=jnp.float32)
           + b_ref[0] + h_ref[0] * hc_ref[0])
    mu = jnp.mean(out, axis=-1, keepdims=True)
    var = jnp.mean((out - mu) ** 2, axis=-1, keepdims=True)
    o_ref[0] = (out - mu) / jnp.sqrt(var + 1e-5) * g_ref[0] + bb_ref[0]


def _epilogue(tp, den, h, w, b, hc, g, bb, expand):
    vec = lambda: pl.BlockSpec((1, 1, OUT_DIM), lambda i, j: (i, 0, 0))
    scrow = lambda w_: pl.BlockSpec(
        (_QR, w_), lambda i, j: ((1 - i) * (_NPAD // _QR) + j, 0))
    return pl.pallas_call(
        _epi_body,
        grid=(2, _NPAD // _QR),
        in_specs=[
            scrow(OUT_DIM),
            scrow(H),
            pl.BlockSpec((1, _QR, OUT_DIM), lambda i, j: (i, j, 0)),
            pl.BlockSpec((1, OUT_DIM, OUT_DIM), lambda i, j: (i, 0, 0)),
            vec(), vec(), vec(), vec(),
            pl.BlockSpec((H, OUT_DIM), lambda i, j: (0, 0)),
        ],
        out_specs=pl.BlockSpec((1, _QR, OUT_DIM), lambda i, j: (i, j, 0)),
        out_shape=jax.ShapeDtypeStruct((2, _NPAD, OUT_DIM), jnp.float32),
    )(tp, den, h, w, b, hc, g, bb, expand)


def _sc_body(q2, k2, v2, src2, dst2, tp_out, den_out,
             krows0, krows1, qm0, qm1, vrows, exd0, exd1,
             srcl0, srcl1, dstl0, dstl1, srcg0, srcg1,
             dstg0, dstg1, dsts0, dsts1, rottab_ref,
             t_acc, den_acc, gsem, vsem, isem, ssem):
    c = lax.axis_index("c")
    s = lax.axis_index("s")
    zeros16 = jnp.zeros((16,), jnp.float32)
    iota16 = lax.iota(jnp.int32, 16)
    krows = (krows0, krows1)
    qm = (qm0, qm1)
    exd = (exd0, exd1)
    srcl = (srcl0, srcl1)
    dstl = (dstl0, dstl1)
    srcg = (srcg0, srcg1)
    dstg = (dstg0, dstg1)
    dsts = (dsts0, dsts1)
    rottab = rottab_ref

    koff = c * _NPAD
    qoff = (1 - c) * _NPAD
    ebase = c * _EPAD + s * _EPT
    r0 = s * _RPT

    def zbuf_body(r, _):
        for j in range(OUT_DIM // 16):
            qm1[r, pl.ds(j * 16, 16)] = zeros16
        return 0
    lax.fori_loop(0, _C, zbuf_body, 0)

    def zdsts_body(r, _):
        dsts1[pl.ds(r * 16, 16)] = jnp.zeros((16,), jnp.int32)
        return 0
    lax.fori_loop(0, _C // 16, zdsts_body, 0)

    for d in range(DK):
        rottab_ref[d] = (iota16 + d) & 15

    def zexd2_body(g, _):
        cols = jnp.zeros((16,), jnp.float32)
        rows = iota16 + g * 16
        for h in range(H):
            plsc.store_scatter(exd1, [rows, jnp.full((16,), h, jnp.int32)],
                               cols)
        return 0
    lax.fori_loop(0, _C // 16, zexd2_body, 0)

    for kblk in range(_RPT // _C):
        pltpu.sync_copy(qm1, t_acc.at[pl.ds(r0 + kblk * _C, _C)])
        pltpu.sync_copy(exd1, den_acc.at[pl.ds(r0 + kblk * _C, _C)])
    plsc.subcore_barrier()

    def adjust(dst_sg, dst_dg, src_sl, src_dl):
        for j in range(_C // 16):
            sl = pl.ds(j * 16, 16)
            dst_sg[sl] = src_sl[sl] + koff
            dst_dg[sl] = src_dl[sl] + qoff

    pltpu.sync_copy(src2.at[pl.ds(ebase, _C)], srcl0)
    pltpu.sync_copy(dst2.at[pl.ds(ebase, _C)], dstl0)
    adjust(srcg0, dstg0, srcl0, dstl0)
    pltpu.async_copy(k2.at[srcg0], krows0, gsem)
    pltpu.async_copy(q2.at[dstg0], qm0, gsem)
    pltpu.async_copy(src2.at[pl.ds(ebase + _C, _C)], srcl1, isem)
    pltpu.async_copy(dst2.at[pl.ds(ebase + _C, _C)], dstl1, isem)
    pltpu.async_copy(qm1, t_acc.at[dsts1], ssem, add=True)
    pltpu.async_copy(exd1, den_acc.at[dsts1], ssem, add=True)

    def score_phase(kr, qr, xd):
        @plsc.parallel_loop(0, _C // 16, unroll=1)
        def _score_groups(g):
            rows = iota16 + g * 16
            for h in range(H):
                acc0 = zeros16
                acc1 = zeros16
                for half in range(2):
                    qvs = []
                    kvs = []
                    for dd in range(DK // 2):
                        d = half * (DK // 2) + dd
                        col = rottab[d] + h * DK
                        qvs.append(plsc.load_gather(qr, [rows, col]))
                        kvs.append(plsc.load_gather(kr, [rows, col]))
                    for dd in range(DK // 2):
                        if dd % 2 == 0:
                            acc0 = acc0 + qvs[dd] * kvs[dd]
                        else:
                            acc1 = acc1 + qvs[dd] * kvs[dd]
                ex = jnp.exp(jnp.minimum(acc0 + acc1, 60.0))
                plsc.store_scatter(xd, [rows, jnp.full((16,), h, jnp.int32)],
                                   ex)

    def m_phase(qr, xd):
        @plsc.parallel_loop(0, _C, unroll=2)
        def _m_edges(e):
            erow = jnp.full((16,), e, jnp.int32)
            for h in range(H):
                exb = plsc.load_gather(xd, [erow,
                                            jnp.full((16,), h, jnp.int32)])
                vv = vrows[e, pl.ds(h * DK, DK)]
                qr[e, pl.ds(h * DK, DK)] = vv * exb

    def drain(sem, src, dst):
        pltpu.make_async_copy(src, dst, sem).wait()

    def pair_body(i, _):
        for b in range(2):
            ci = i * 2 + b
            o = 1 - b
            drain(isem, src2.at[pl.ds(0, _C)], srcl[o])
            drain(isem, dst2.at[pl.ds(0, _C)], dstl[o])
            adjust(srcg[o], dstg[o], srcl[o], dstl[o])
            drain(ssem, k2.at[pl.ds(0, _C)], qm[o])
            drain(ssem, den_out.at[pl.ds(0, _C)], exd[o])
            drain(gsem, k2.at[pl.ds(0, _C)], krows[b])
            drain(gsem, q2.at[pl.ds(0, _C)], qm[b])
            pltpu.async_copy(k2.at[srcg[o]], krows[o], gsem)
            pltpu.async_copy(q2.at[dstg[o]], qm[o], gsem)
            for j in range(_C // 16):
                sl = pl.ds(j * 16, 16)
                dsts[b][sl] = dstl[b][sl]
            base2 = ebase + (ci + 2) * _C
            pltpu.async_copy(src2.at[pl.ds(base2, _C)], srcl[b], isem)
            pltpu.async_copy(dst2.at[pl.ds(base2, _C)], dstl[b], isem)
            pltpu.async_copy(v2.at[srcg[b]], vrows, vsem)
            score_phase(krows[b], qm[b], exd[b])
            drain(vsem, v2.at[pl.ds(0, _C)], vrows)
            m_phase(qm[b], exd[b])
            pltpu.async_copy(qm[b], t_acc.at[dsts[b]], ssem, add=True)
            pltpu.async_copy(exd[b], den_acc.at[dsts[b]], ssem, add=True)
        return 0

    lax.fori_loop(0, _NCHUNK // 2, pair_body, 0)

    drain(ssem, k2.at[pl.ds(0, _C)], qm1)
    drain(ssem, den_out.at[pl.ds(0, _C)], exd1)
    drain(gsem, k2.at[pl.ds(0, _C)], krows0)
    drain(gsem, q2.at[pl.ds(0, _C)], qm0)
    drain(isem, src2.at[pl.ds(0, _C)], srcl0)
    drain(isem, dst2.at[pl.ds(0, _C)], dstl0)
    plsc.subcore_barrier()

    out_base = c * _NPAD + r0
    pltpu.sync_copy(t_acc.at[pl.ds(r0, _RPT)],
                    tp_out.at[pl.ds(out_base, _RPT)])
    pltpu.sync_copy(den_acc.at[pl.ds(r0, _RPT)],
                    den_out.at[pl.ds(out_base, _RPT)])


def _sc_middle(q2, k2, v2, src2, dst2):
    mesh = plsc.VectorSubcoreMesh(core_axis_name="c", subcore_axis_name="s")
    f = pl.kernel(
        _sc_body,
        mesh=mesh,
        compiler_params=pltpu.CompilerParams(needs_layout_passes=False,
                                             use_tc_tiling_on_sc=False),
        out_type=[
            jax.ShapeDtypeStruct((2 * _NPAD, OUT_DIM), jnp.float32),
            jax.ShapeDtypeStruct((2 * _NPAD, H), jnp.float32),
        ],
        scratch_types=[
            pltpu.VMEM((_C, OUT_DIM), jnp.float32),
            pltpu.VMEM((_C, OUT_DIM), jnp.float32),
            pltpu.VMEM((_C, OUT_DIM), jnp.float32),
            pltpu.VMEM((_C, OUT_DIM), jnp.float32),
            pltpu.VMEM((_C, OUT_DIM), jnp.float32),
            pltpu.VMEM((_C, H), jnp.float32),
            pltpu.VMEM((_C, H), jnp.float32),
            pltpu.VMEM((_C,), jnp.int32),
            pltpu.VMEM((_C,), jnp.int32),
            pltpu.VMEM((_C,), jnp.int32),
            pltpu.VMEM((_C,), jnp.int32),
            pltpu.VMEM((_C,), jnp.int32),
            pltpu.VMEM((_C,), jnp.int32),
            pltpu.VMEM((_C,), jnp.int32),
            pltpu.VMEM((_C,), jnp.int32),
            pltpu.VMEM((_C,), jnp.int32),
            pltpu.VMEM((_C,), jnp.int32),
            pltpu.VMEM((DK, 16), jnp.int32),
            pltpu.VMEM_SHARED((_NPAD, OUT_DIM), jnp.float32),
            pltpu.VMEM_SHARED((_NPAD, H), jnp.float32),
            pltpu.SemaphoreType.DMA,
            pltpu.SemaphoreType.DMA,
            pltpu.SemaphoreType.DMA,
            pltpu.SemaphoreType.DMA,
        ],
    )
    return f(q2, k2, v2, src2, dst2)


def kernel(h_author, h_paper, edge_index_writes, edge_index_written_by,
           Wk_author, bk_author, Wq_author, bq_author, Wv_author, bv_author,
           Wa_author, ba_author, ln_g_author, ln_b_author, skip_author,
           Wk_paper, bk_paper, Wq_paper, bq_paper, Wv_paper, bv_paper,
           Wa_paper, ba_paper, ln_g_paper, ln_b_paper, skip_paper,
           pri_writes, att_writes, msg_writes,
           pri_written_by, att_written_by, msg_written_by):
    scale_w = pri_writes / math.sqrt(DK)
    scale_wb = pri_written_by / math.sqrt(DK)
    bd_att_w = _block_diag(att_writes * scale_w[:, None, None])
    bd_msg_w = _block_diag(msg_writes)
    bd_att_wb = _block_diag(att_written_by * scale_wb[:, None, None])
    bd_msg_wb = _block_diag(msg_written_by)
    w_author = jnp.concatenate(
        [Wq_author, Wk_author @ bd_att_w, Wv_author @ bd_msg_w], axis=1)
    b_author = jnp.concatenate(
        [bq_author, bk_author @ bd_att_w, bv_author @ bd_msg_w])[None, :]
    w_paper = jnp.concatenate(
        [Wq_paper, Wk_paper @ bd_att_wb, Wv_paper @ bd_msg_wb], axis=1)
    b_paper = jnp.concatenate(
        [bq_paper, bk_paper @ bd_att_wb, bv_paper @ bd_msg_wb])[None, :]

    x = jnp.pad(jnp.stack([h_author, h_paper]),
                ((0, 0), (0, _NPAD - N_NODE), (0, 0)))
    w = jnp.stack([w_author, w_paper])
    b = jnp.stack([b_author, b_paper])

    q2, k2, v2 = _qkv_project(x, w, b)

    npad_e = _EPAD - E
    pad_src = jnp.zeros((npad_e,), jnp.int32)
    pad_dst = jnp.full((npad_e,), N_NODE, jnp.int32)
    tail = jnp.zeros((4 * _C,), jnp.int32)
    src2 = jnp.concatenate([edge_index_writes[0], pad_src,
                            edge_index_written_by[0], pad_src, tail])
    dst2 = jnp.concatenate([edge_index_writes[1], pad_dst,
                            edge_index_written_by[1], pad_dst, tail])

    tp, den = _sc_middle(q2, k2, v2, src2, dst2)

    alpha_a = jax.nn.sigmoid(skip_author)[0]
    alpha_p = jax.nn.sigmoid(skip_paper)[0]
    w2 = jnp.stack([Wa_author * alpha_a, Wa_paper * alpha_p])
    b2 = jnp.stack([(ba_author * alpha_a)[None, :], (ba_paper * alpha_p)[None, :]])
    hc = jnp.stack([jnp.full((1, OUT_DIM), 1.0 - alpha_a, jnp.float32),
                    jnp.full((1, OUT_DIM), 1.0 - alpha_p, jnp.float32)])
    g2 = jnp.stack([ln_g_author[None, :], ln_g_paper[None, :]])
    bb2 = jnp.stack([ln_b_author[None, :], ln_b_paper[None, :]])
    expand = jnp.kron(jnp.eye(H, dtype=jnp.float32),
                      jnp.ones((1, DK), jnp.float32))
    out = _epilogue(tp, den, x, w2, b2, hc, g2, bb2, expand)
    return (out[0, :N_NODE], out[1, :N_NODE])
